# Initial kernel scaffold; baseline (speedup 1.0000x reference)
import jax
import jax.numpy as jnp
from jax.experimental import pallas as pl


def kernel(x, edge_index_connections, edge_index_destinations, W1l, W1r, W2l, W2r, W3l, W3r, W4l, W4r, g1, b1, g2, b2, g3, b3, g4, b4):
    raise NotImplementedError("write your pallas kernel here")


# speedup vs baseline: 4.3556x; 4.3556x over previous
import functools

import jax
import jax.numpy as jnp
from jax import lax
from jax.experimental import pallas as pl
from jax.experimental.pallas import tpu as pltpu
from jax.experimental.pallas import tpu_sc as plsc

N = 10000
H = 128
NP = 10240
NW = 32
ROWS_PER_TILE = NP // 16
CHUNK = 128


def _make_sc_agg(n_chunks):
    mesh = plsc.VectorSubcoreMesh(core_axis_name="c", subcore_axis_name="s")

    @functools.partial(
        pl.kernel,
        out_type=[
            jax.ShapeDtypeStruct((2, NP, H), jnp.float32),
            jax.ShapeDtypeStruct((2, NP), jnp.float32),
        ],
        mesh=mesh,
        scratch_types=[
            pltpu.VMEM((n_chunks, CHUNK), jnp.int32),
            pltpu.VMEM((n_chunks, CHUNK), jnp.int32),
            pltpu.VMEM((CHUNK, H), jnp.float32),
            pltpu.VMEM((CHUNK,), jnp.float32),
            pltpu.VMEM((ROWS_PER_TILE,), jnp.float32),
            pltpu.VMEM_SHARED((NP, H), jnp.float32),
            pltpu.VMEM_SHARED((NP,), jnp.float32),
            pltpu.SemaphoreType.DMA,
        ],
    )
    def sc_agg(src_hbm, dst_hbm, h_hbm, agg_out, cnt_out,
               src_v, dst_v, rows_v, ones_v, zeros_v, agg_s, cnt_s, sem):
        cid = lax.axis_index("c")
        sid = lax.axis_index("s")
        wid = cid * 16 + sid
        base = sid * ROWS_PER_TILE

        pltpu.sync_copy(src_hbm.at[wid], src_v)
        pltpu.sync_copy(dst_hbm.at[wid], dst_v)

        def _fill_rows(i, _):
            for j in range(H // 16):
                rows_v[i, pl.ds(j * 16, 16)] = jnp.zeros((16,), jnp.float32)
            return 0
        lax.fori_loop(0, CHUNK, _fill_rows, 0)
        for j in range(CHUNK // 16):
            ones_v[pl.ds(j * 16, 16)] = jnp.ones((16,), jnp.float32)
        for j in range(ROWS_PER_TILE // 16):
            zeros_v[pl.ds(j * 16, 16)] = jnp.zeros((16,), jnp.float32)

        for k in range(ROWS_PER_TILE // CHUNK):
            pltpu.sync_copy(rows_v, agg_s.at[pl.ds(base + k * CHUNK, CHUNK)])
        pltpu.sync_copy(zeros_v, cnt_s.at[pl.ds(base, ROWS_PER_TILE)])
        plsc.subcore_barrier()

        def body(j, _):
            pltpu.async_copy(h_hbm.at[src_v.at[j]], rows_v, sem).wait()
            pltpu.sync_copy(rows_v, agg_s.at[dst_v.at[j]], add=True)
            pltpu.sync_copy(ones_v, cnt_s.at[dst_v.at[j]], add=True)
            return 0
        lax.fori_loop(0, n_chunks, body, 0)
        plsc.subcore_barrier()

        for k in range(ROWS_PER_TILE // CHUNK):
            sl = pl.ds(base + k * CHUNK, CHUNK)
            pltpu.sync_copy(agg_s.at[sl], agg_out.at[cid].at[sl])
        pltpu.sync_copy(cnt_s.at[pl.ds(base, ROWS_PER_TILE)],
                        cnt_out.at[cid].at[pl.ds(base, ROWS_PER_TILE)])

    return sc_agg


_N_CHUNKS_CONN = -(-320000 // (NW * CHUNK))
_N_CHUNKS_DEST = -(-10000 // (NW * CHUNK))
_sc_agg_conn = _make_sc_agg(_N_CHUNKS_CONN)
_sc_agg_dest = _make_sc_agg(_N_CHUNKS_DEST)


def _tc_layer_body(agg_ref, cinv_ref, h_ref, wl_ref, wr_ref, g_ref, b_ref,
                   out_ref):
    agg = agg_ref[0] + agg_ref[1]
    mean = agg * cinv_ref[...]
    y = jnp.dot(mean, wl_ref[...], preferred_element_type=jnp.float32)
    y = y + jnp.dot(h_ref[...], wr_ref[...], preferred_element_type=jnp.float32)
    ss = jnp.sum(y * y, axis=1, keepdims=True)
    y = y / jnp.maximum(jnp.sqrt(ss), 1e-12)
    y = jnp.maximum(y, 0.0)
    valid = lax.broadcasted_iota(jnp.int32, (NP, 1), 0) < N
    yz = jnp.where(valid, y, 0.0)
    mu = jnp.sum(yz, axis=0, keepdims=True) * (1.0 / N)
    d = jnp.where(valid, y - mu, 0.0)
    var = jnp.sum(d * d, axis=0, keepdims=True) * (1.0 / N)
    o = g_ref[...] * ((y - mu) * lax.rsqrt(var + 1e-5)) + b_ref[...]
    out_ref[...] = jnp.where(valid, o, 0.0)


_tc_layer = pl.pallas_call(
    _tc_layer_body,
    out_shape=jax.ShapeDtypeStruct((NP, H), jnp.float32),
)


def _prep_edges(edge_index, n_chunks):
    total = NW * n_chunks * CHUNK
    pad = total - edge_index.shape[1]
    src = jnp.concatenate(
        [edge_index[0], jnp.full((pad,), N, jnp.int32)]).reshape(NW, n_chunks, CHUNK)
    dst = jnp.concatenate(
        [edge_index[1], jnp.full((pad,), N, jnp.int32)]).reshape(NW, n_chunks, CHUNK)
    return src, dst


def kernel(x, edge_index_connections, edge_index_destinations,
           W1l, W1r, W2l, W2r, W3l, W3r, W4l, W4r,
           g1, b1, g2, b2, g3, b3, g4, b4):
    src_c, dst_c = _prep_edges(edge_index_connections, _N_CHUNKS_CONN)
    src_d, dst_d = _prep_edges(edge_index_destinations, _N_CHUNKS_DEST)
    h = jnp.concatenate([x, jnp.zeros((NP - N, H), jnp.float32)], axis=0)

    def layer(h, src, dst, sc_agg, Wl, Wr, g, b):
        agg, cnt = sc_agg(src, dst, h)
        cinv = (1.0 / jnp.maximum(cnt[0] + cnt[1], 1.0))[:, None]
        return _tc_layer(agg, cinv, h, Wl, Wr,
                         g.reshape(1, H), b.reshape(1, H))

    h = layer(h, src_c, dst_c, _sc_agg_conn, W1l, W1r, g1, b1)
    for _ in range(3):
        h = layer(h, src_c, dst_c, _sc_agg_conn, W4l, W4r, g2, b2)
    h = layer(h, src_d, dst_d, _sc_agg_dest, W2l, W2r, g3, b3)
    for _ in range(2):
        h = layer(h, src_c, dst_c, _sc_agg_conn, W3l, W3r, g4, b4)
    return h[:N]
